# baseline (device time: 11693 ns/iter reference)
import jax
import jax.numpy as jnp
from jax import lax
from jax.experimental import pallas as pl
from jax.experimental.pallas import tpu as pltpu

C = 64
MAX_CHUNKS = 8


def kernel(x, dest):
    m, n = x.shape
    my_y = lax.axis_index("y")

    keep = dest == my_y
    ki = keep.astype(jnp.int32)
    c_keep = jnp.cumsum(ki)
    c_peer = jnp.arange(1, m + 1, dtype=jnp.int32) - c_keep
    k = c_keep[-1]
    p = m - k
    dst_pos = jnp.where(keep, p + c_keep - 1, c_peer - 1)
    dp_2d = jnp.reshape(dst_pos, (1, m))
    k_arr = jnp.reshape(k, (1,)).astype(jnp.int32)

    def body(k_ref, dp_ref, x_ref, out_ref, xs_ref, recv_ref,
             send_sems, recv_sems):
        my_x = lax.axis_index("x")
        yy = lax.axis_index("y")
        peer = (my_x, 1 - yy)
        k_ = k_ref[0]
        p_ = m - k_
        n_ch = (p_ + C - 1) // C

        barrier = pltpu.get_barrier_semaphore()
        pl.semaphore_signal(
            barrier, inc=1, device_id=peer,
            device_id_type=pl.DeviceIdType.MESH,
        )

        row_i = lax.broadcasted_iota(jnp.int32, (m, m), 0)
        perm = (row_i == dp_ref[...]).astype(jnp.bfloat16)
        xv = x_ref[...]
        x_hi = xv.astype(jnp.bfloat16)
        x_lo = (xv - x_hi.astype(jnp.float32)).astype(jnp.bfloat16)
        xs = jnp.dot(
            perm, x_hi, preferred_element_type=jnp.float32
        ) + jnp.dot(perm, x_lo, preferred_element_type=jnp.float32)
        xs_ref[...] = xs

        pl.semaphore_wait(barrier, 1)

        def chunk(c):
            off = pl.multiple_of(c * C, C)
            return pltpu.make_async_remote_copy(
                src_ref=xs_ref.at[pl.ds(off, C)],
                dst_ref=recv_ref.at[pl.ds(off, C)],
                send_sem=send_sems.at[c],
                recv_sem=recv_sems.at[c],
                device_id=peer,
                device_id_type=pl.DeviceIdType.MESH,
            )

        def issue(c, _):
            chunk(c).start()
            return _
        lax.fori_loop(0, n_ch, issue, None)

        def wait_in(c, _):
            chunk(c).wait_recv()
            return _
        lax.fori_loop(0, n_ch, wait_in, None)

        rv = recv_ref[...]
        row = lax.broadcasted_iota(jnp.int32, (m, n), 0)
        z = jnp.where(row < p_, rv, xs)

        @pl.when(yy == 0)
        def _():
            out_ref[...] = pltpu.roll(z, k_, 0)

        @pl.when(yy != 0)
        def _():
            out_ref[...] = z

        def wait_out(c, _):
            chunk(c).wait_send()
            return _
        lax.fori_loop(0, n_ch, wait_out, None)

    return pl.pallas_call(
        body,
        out_shape=jax.ShapeDtypeStruct((m, n), x.dtype),
        in_specs=[
            pl.BlockSpec(memory_space=pltpu.SMEM),
            pl.BlockSpec(memory_space=pltpu.VMEM),
            pl.BlockSpec(memory_space=pltpu.VMEM),
        ],
        out_specs=pl.BlockSpec(memory_space=pltpu.VMEM),
        scratch_shapes=[
            pltpu.VMEM((m, n), x.dtype),
            pltpu.VMEM((m, n), x.dtype),
            pltpu.SemaphoreType.DMA((MAX_CHUNKS,)),
            pltpu.SemaphoreType.DMA((MAX_CHUNKS,)),
        ],
        compiler_params=pltpu.CompilerParams(collective_id=0),
    )(k_arr, dp_2d, x)


# device time: 10079 ns/iter; 1.1601x vs baseline; 1.1601x over previous
import jax
import jax.numpy as jnp
from jax import lax
from jax.experimental import pallas as pl
from jax.experimental.pallas import tpu as pltpu

C = 64
MAX_CHUNKS = 8


def kernel(x, dest):
    m, n = x.shape
    dest2d = jnp.reshape(dest, (1, m))

    def body(d_ref, x_ref, out_ref, xs_ref, recv_ref,
             send_sems, recv_sems):
        my_x = lax.axis_index("x")
        yy = lax.axis_index("y")
        peer = (my_x, 1 - yy)

        barrier = pltpu.get_barrier_semaphore()
        pl.semaphore_signal(
            barrier, inc=1, device_id=peer,
            device_id_type=pl.DeviceIdType.MESH,
        )

        keep = d_ref[...] == yy
        k_ = jnp.sum(keep.astype(jnp.int32))
        p_ = m - k_
        n_ch = (p_ + C - 1) // C

        io0 = lax.broadcasted_iota(jnp.int32, (m, m), 0)
        io1 = lax.broadcasted_iota(jnp.int32, (m, m), 1)
        utri = (io0 <= io1).astype(jnp.bfloat16)
        c_keep = jnp.dot(
            keep.astype(jnp.bfloat16), utri,
            preferred_element_type=jnp.float32,
        ).astype(jnp.int32)
        pos1 = lax.broadcasted_iota(jnp.int32, (1, m), 1) + 1
        c_peer = pos1 - c_keep
        dst_pos = jnp.where(keep, p_ + c_keep - 1, c_peer - 1)

        perm = (io0 == dst_pos).astype(jnp.bfloat16)
        xv = x_ref[...]
        x_hi = xv.astype(jnp.bfloat16)
        x_lo = (xv - x_hi.astype(jnp.float32)).astype(jnp.bfloat16)
        xs = jnp.dot(
            perm, x_hi, preferred_element_type=jnp.float32
        ) + jnp.dot(perm, x_lo, preferred_element_type=jnp.float32)
        xs_ref[...] = xs

        pl.semaphore_wait(barrier, 1)

        def chunk(c):
            off = pl.multiple_of(c * C, C)
            return pltpu.make_async_remote_copy(
                src_ref=xs_ref.at[pl.ds(off, C)],
                dst_ref=recv_ref.at[pl.ds(off, C)],
                send_sem=send_sems.at[c],
                recv_sem=recv_sems.at[c],
                device_id=peer,
                device_id_type=pl.DeviceIdType.MESH,
            )

        def issue(c, _):
            chunk(c).start()
            return _
        lax.fori_loop(0, n_ch, issue, None)

        def wait_in(c, _):
            chunk(c).wait_recv()
            return _
        lax.fori_loop(0, n_ch, wait_in, None)

        rv = recv_ref[...]
        row = lax.broadcasted_iota(jnp.int32, (m, n), 0)
        z = jnp.where(row < p_, rv, xs)

        @pl.when(yy == 0)
        def _():
            out_ref[...] = pltpu.roll(z, k_, 0)

        @pl.when(yy != 0)
        def _():
            out_ref[...] = z

        def wait_out(c, _):
            chunk(c).wait_send()
            return _
        lax.fori_loop(0, n_ch, wait_out, None)

    return pl.pallas_call(
        body,
        out_shape=jax.ShapeDtypeStruct((m, n), x.dtype),
        in_specs=[
            pl.BlockSpec(memory_space=pltpu.VMEM),
            pl.BlockSpec(memory_space=pltpu.VMEM),
        ],
        out_specs=pl.BlockSpec(memory_space=pltpu.VMEM),
        scratch_shapes=[
            pltpu.VMEM((m, n), x.dtype),
            pltpu.VMEM((m, n), x.dtype),
            pltpu.SemaphoreType.DMA((MAX_CHUNKS,)),
            pltpu.SemaphoreType.DMA((MAX_CHUNKS,)),
        ],
        compiler_params=pltpu.CompilerParams(collective_id=0),
    )(dest2d, x)
